# baseline (speedup 1.0000x reference)
import jax
import jax.numpy as jnp
from jax.experimental import pallas as pl
from jax.experimental.pallas import tpu as pltpu

N_FEAT = 128
N_CLASSES = 100000
N_ROWS = 4096
SCALE = 30.0
C_TILE = 512
N_CTILES = pl.cdiv(N_CLASSES, C_TILE)


def _fused_kernel(x_ref, tgt_ref, lut_ref, logits_ref, loss_ref,
                  m_ref, s_ref, t_ref):
    j = pl.program_id(0)
    x = x_ref[...] * SCALE
    w = lut_ref[...]
    logits = jax.lax.dot_general(
        x, w, (((1,), (1,)), ((), ())),
        preferred_element_type=jnp.float32)

    cols = j * C_TILE + jax.lax.broadcasted_iota(
        jnp.int32, (N_ROWS, C_TILE), 1)

    @pl.when(j == N_CTILES - 1)
    def _mask_ragged():
        logits_ref[...] = jnp.where(cols < N_CLASSES, logits, -jnp.inf)

    @pl.when(j != N_CTILES - 1)
    def _store_full():
        logits_ref[...] = logits

    lg = logits_ref[...]
    tile_max = jnp.max(lg, axis=1, keepdims=True)
    hit = cols == tgt_ref[...]
    tpart = jnp.sum(jnp.where(hit, lg, 0.0), axis=1, keepdims=True)

    @pl.when(j == 0)
    def _init():
        m_ref[...] = tile_max
        s_ref[...] = jnp.sum(jnp.exp(lg - tile_max), axis=1, keepdims=True)
        t_ref[...] = tpart

    @pl.when(j > 0)
    def _update():
        m_old = m_ref[...]
        m_new = jnp.maximum(m_old, tile_max)
        s_ref[...] = (s_ref[...] * jnp.exp(m_old - m_new)
                      + jnp.sum(jnp.exp(lg - m_new), axis=1, keepdims=True))
        m_ref[...] = m_new
        t_ref[...] = t_ref[...] + tpart

    @pl.when(j == N_CTILES - 1)
    def _finalize():
        nll = m_ref[...] + jnp.log(s_ref[...]) - t_ref[...]
        loss_ref[0, 0] = jnp.sum(nll) / N_ROWS


def kernel(inputs, targets, lut):
    tgt2d = targets.astype(jnp.int32).reshape(N_ROWS, 1)
    logits, loss = pl.pallas_call(
        _fused_kernel,
        grid=(N_CTILES,),
        in_specs=[
            pl.BlockSpec((N_ROWS, N_FEAT), lambda j: (0, 0)),
            pl.BlockSpec((N_ROWS, 1), lambda j: (0, 0)),
            pl.BlockSpec((C_TILE, N_FEAT), lambda j: (j, 0)),
        ],
        out_specs=[
            pl.BlockSpec((N_ROWS, C_TILE), lambda j: (0, j)),
            pl.BlockSpec(memory_space=pltpu.SMEM),
        ],
        out_shape=[
            jax.ShapeDtypeStruct((N_ROWS, N_CLASSES), jnp.float32),
            jax.ShapeDtypeStruct((1, 1), jnp.float32),
        ],
        scratch_shapes=[
            pltpu.VMEM((N_ROWS, 1), jnp.float32),
            pltpu.VMEM((N_ROWS, 1), jnp.float32),
            pltpu.VMEM((N_ROWS, 1), jnp.float32),
        ],
        compiler_params=pltpu.CompilerParams(
            dimension_semantics=("arbitrary",)),
    )(inputs, tgt2d, lut)
    return loss[0, 0], logits


# speedup vs baseline: 1.1228x; 1.1228x over previous
import functools

import jax
import jax.numpy as jnp
from jax import lax
from jax.experimental import pallas as pl
from jax.experimental.pallas import tpu as pltpu
from jax.experimental.pallas import tpu_sc as plsc

N_FEAT = 128
N_CLASSES = 100000
N_ROWS = 4096
SCALE = 30.0
C_TILE = 512
N_CTILES = pl.cdiv(N_CLASSES, C_TILE)

_NC = 2
_NS = 16
_NW = _NC * _NS
_B_PER_W = N_ROWS // _NW


def _sc_gather(lut_hbm, tgt_hbm, out_hbm, idx_v, rows_v, sem):
    wid = lax.axis_index("s") * _NC + lax.axis_index("c")
    base = wid * _B_PER_W
    pltpu.sync_copy(tgt_hbm.at[pl.ds(base, _B_PER_W)], idx_v)
    pltpu.async_copy(lut_hbm.at[idx_v], rows_v, sem).wait()
    pltpu.sync_copy(rows_v, out_hbm.at[pl.ds(base, _B_PER_W)])


def _gather_rows(lut, targets):
    mesh = plsc.VectorSubcoreMesh(core_axis_name="c", subcore_axis_name="s")
    return functools.partial(
        pl.kernel,
        mesh=mesh,
        out_type=jax.ShapeDtypeStruct((N_ROWS, N_FEAT), jnp.float32),
        scratch_types=[
            pltpu.VMEM((_B_PER_W,), jnp.int32),
            pltpu.VMEM((_B_PER_W, N_FEAT), jnp.float32),
            pltpu.SemaphoreType.DMA,
        ],
    )(_sc_gather)(lut, targets)


def _fused_kernel(x_ref, g_ref, lut_ref, logits_ref, loss_ref, m_ref, s_ref):
    j = pl.program_id(0)
    x = x_ref[...] * SCALE
    w = lut_ref[...]
    logits = jax.lax.dot_general(
        x, w, (((1,), (1,)), ((), ())),
        preferred_element_type=jnp.float32)

    @pl.when(j == 0)
    def _init():
        m_ref[...] = jnp.full((N_ROWS, 1), -jnp.inf, jnp.float32)
        s_ref[...] = jnp.zeros((N_ROWS, N_FEAT), jnp.float32)

    def _accumulate(lg):
        logits_ref[...] = lg
        m_old = m_ref[...]
        m_new = jnp.maximum(m_old, jnp.max(lg, axis=1, keepdims=True))
        e = jnp.exp(lg - m_new)
        part = ((e[:, 0:128] + e[:, 128:256])
                + (e[:, 256:384] + e[:, 384:512]))
        s_ref[...] = s_ref[...] * jnp.exp(m_old - m_new) + part
        m_ref[...] = m_new

    @pl.when(j != N_CTILES - 1)
    def _full_tile():
        _accumulate(logits)

    @pl.when(j == N_CTILES - 1)
    def _ragged_tile():
        cols = j * C_TILE + jax.lax.broadcasted_iota(
            jnp.int32, (N_ROWS, C_TILE), 1)
        _accumulate(jnp.where(cols < N_CLASSES, logits, -jnp.inf))
        s_row = jnp.sum(s_ref[...], axis=1, keepdims=True)
        tgt = jnp.sum(x * g_ref[...], axis=1, keepdims=True)
        nll = m_ref[...] + jnp.log(s_row) - tgt
        loss_ref[0, 0] = jnp.sum(nll) / N_ROWS


def kernel(inputs, targets, lut):
    g_rows = _gather_rows(lut, targets.astype(jnp.int32))
    logits, loss = pl.pallas_call(
        _fused_kernel,
        grid=(N_CTILES,),
        in_specs=[
            pl.BlockSpec((N_ROWS, N_FEAT), lambda j: (0, 0)),
            pl.BlockSpec((N_ROWS, N_FEAT), lambda j: (0, 0)),
            pl.BlockSpec((C_TILE, N_FEAT), lambda j: (j, 0)),
        ],
        out_specs=[
            pl.BlockSpec((N_ROWS, C_TILE), lambda j: (0, j)),
            pl.BlockSpec(memory_space=pltpu.SMEM),
        ],
        out_shape=[
            jax.ShapeDtypeStruct((N_ROWS, N_CLASSES), jnp.float32),
            jax.ShapeDtypeStruct((1, 1), jnp.float32),
        ],
        scratch_shapes=[
            pltpu.VMEM((N_ROWS, 1), jnp.float32),
            pltpu.VMEM((N_ROWS, N_FEAT), jnp.float32),
        ],
        compiler_params=pltpu.CompilerParams(
            dimension_semantics=("arbitrary",)),
    )(inputs, g_rows, lut)
    return loss[0, 0], logits


# speedup vs baseline: 1.2436x; 1.1076x over previous
import functools

import jax
import jax.numpy as jnp
from jax import lax
from jax.experimental import pallas as pl
from jax.experimental.pallas import tpu as pltpu
from jax.experimental.pallas import tpu_sc as plsc

N_FEAT = 128
N_CLASSES = 100000
N_ROWS = 4096
SCALE = 30.0
R_TILE = 2048
C_TILE = 1024
N_RTILES = N_ROWS // R_TILE
N_CTILES = pl.cdiv(N_CLASSES, C_TILE)
N_CHUNKS = C_TILE // N_FEAT

_NC = 2
_NS = 16
_NW = _NC * _NS
_B_PER_W = N_ROWS // _NW


def _sc_gather(lut_hbm, tgt_hbm, out_hbm, idx_v, rows_v, sem):
    wid = lax.axis_index("s") * _NC + lax.axis_index("c")
    base = wid * _B_PER_W
    pltpu.sync_copy(tgt_hbm.at[pl.ds(base, _B_PER_W)], idx_v)
    pltpu.async_copy(lut_hbm.at[idx_v], rows_v, sem).wait()
    pltpu.sync_copy(rows_v, out_hbm.at[pl.ds(base, _B_PER_W)])


def _gather_rows(lut, targets):
    mesh = plsc.VectorSubcoreMesh(core_axis_name="c", subcore_axis_name="s")
    return functools.partial(
        pl.kernel,
        mesh=mesh,
        out_type=jax.ShapeDtypeStruct((N_ROWS, N_FEAT), jnp.float32),
        scratch_types=[
            pltpu.VMEM((_B_PER_W,), jnp.int32),
            pltpu.VMEM((_B_PER_W, N_FEAT), jnp.float32),
            pltpu.SemaphoreType.DMA,
        ],
    )(_sc_gather)(lut, targets)


def _fused_kernel(x_ref, g_ref, lut_ref, logits_ref, loss_ref, m_ref, s_ref):
    i = pl.program_id(0)
    j = pl.program_id(1)
    x = x_ref[...] * SCALE
    w = lut_ref[...]
    logits = jax.lax.dot_general(
        x, w, (((1,), (1,)), ((), ())),
        preferred_element_type=jnp.float32)

    @pl.when(j == 0)
    def _init():
        m_ref[...] = jnp.full((R_TILE, N_FEAT), -jnp.inf, jnp.float32)
        s_ref[...] = jnp.zeros((R_TILE, N_FEAT), jnp.float32)

    def _accumulate(lg):
        logits_ref[...] = lg
        chunks = [lg[:, k * N_FEAT:(k + 1) * N_FEAT] for k in range(N_CHUNKS)]
        m_old = m_ref[...]
        m_new = m_old
        for c in chunks:
            m_new = jnp.maximum(m_new, c)
        acc = s_ref[...] * jnp.exp(m_old - m_new)
        for c in chunks:
            acc = acc + jnp.exp(c - m_new)
        s_ref[...] = acc
        m_ref[...] = m_new

    @pl.when(j != N_CTILES - 1)
    def _full_tile():
        _accumulate(logits)

    @pl.when(j == N_CTILES - 1)
    def _ragged_tile():
        cols = j * C_TILE + jax.lax.broadcasted_iota(
            jnp.int32, (R_TILE, C_TILE), 1)
        _accumulate(jnp.where(cols < N_CLASSES, logits, -jnp.inf))
        m_l = m_ref[...]
        m_row = jnp.max(m_l, axis=1, keepdims=True)
        s_row = jnp.sum(s_ref[...] * jnp.exp(m_l - m_row),
                        axis=1, keepdims=True)
        tgt = jnp.sum(x * g_ref[...], axis=1, keepdims=True)
        part = jnp.sum(m_row + jnp.log(s_row) - tgt) / N_ROWS

        @pl.when(i == 0)
        def _first():
            loss_ref[0, 0] = part

        @pl.when(i > 0)
        def _rest():
            loss_ref[0, 0] = loss_ref[0, 0] + part


def kernel(inputs, targets, lut):
    g_rows = _gather_rows(lut, targets.astype(jnp.int32))
    logits, loss = pl.pallas_call(
        _fused_kernel,
        grid=(N_RTILES, N_CTILES),
        in_specs=[
            pl.BlockSpec((R_TILE, N_FEAT), lambda i, j: (i, 0)),
            pl.BlockSpec((R_TILE, N_FEAT), lambda i, j: (i, 0)),
            pl.BlockSpec((C_TILE, N_FEAT), lambda i, j: (j, 0)),
        ],
        out_specs=[
            pl.BlockSpec((R_TILE, C_TILE), lambda i, j: (i, j)),
            pl.BlockSpec(memory_space=pltpu.SMEM),
        ],
        out_shape=[
            jax.ShapeDtypeStruct((N_ROWS, N_CLASSES), jnp.float32),
            jax.ShapeDtypeStruct((1, 1), jnp.float32),
        ],
        scratch_shapes=[
            pltpu.VMEM((R_TILE, N_FEAT), jnp.float32),
            pltpu.VMEM((R_TILE, N_FEAT), jnp.float32),
        ],
        compiler_params=pltpu.CompilerParams(
            dimension_semantics=("arbitrary", "arbitrary")),
    )(inputs, g_rows, lut)
    return loss[0, 0], logits
